# baseline (speedup 1.0000x reference)
import functools

import jax
import jax.numpy as jnp
from jax import lax
from jax.experimental import pallas as pl
from jax.experimental.pallas import tpu as pltpu
from jax.experimental.pallas import tpu_sc as plsc

D = 512
LANES = 16
CHUNK = 64


def _pe_add_body(x_hbm, idx_hbm, table_hbm, out_hbm, idx_v, pe_v, x_v, sem):
    nc = 2
    wid = lax.axis_index("s") * nc + lax.axis_index("c")
    n_rows = x_hbm.shape[0]
    rows_per_worker = n_rows // 32
    n_chunks = rows_per_worker // CHUNK

    def chunk_body(ci, carry):
        base = wid * rows_per_worker + ci * CHUNK
        pltpu.sync_copy(idx_hbm.at[pl.ds(base, CHUNK)], idx_v)
        gather = pltpu.async_copy(table_hbm.at[idx_v], pe_v, sem)
        pltpu.sync_copy(x_hbm.at[pl.ds(base, CHUNK)], x_v)
        gather.wait()

        def row_body(r, c2):
            for j in range(D // LANES):
                sl = pl.ds(j * LANES, LANES)
                x_v[r, sl] = x_v[r, sl] + pe_v[r, sl]
            return c2

        lax.fori_loop(0, CHUNK, row_body, 0)
        pltpu.sync_copy(x_v, out_hbm.at[pl.ds(base, CHUNK)])
        return carry

    lax.fori_loop(0, n_chunks, chunk_body, 0)


def kernel(x, pos, img_pe, seq_pe):
    B, L, d_model = x.shape
    n_rows = B * L * 2
    x_rows = x.reshape(n_rows, D)
    table = jnp.concatenate([img_pe, seq_pe], axis=0)
    idx = (pos.astype(jnp.int32) + jnp.array([0, img_pe.shape[0]], jnp.int32)
           ).reshape(n_rows)

    mesh = plsc.VectorSubcoreMesh(core_axis_name="c", subcore_axis_name="s")
    run = pl.kernel(
        _pe_add_body,
        mesh=mesh,
        out_type=jax.ShapeDtypeStruct((n_rows, D), jnp.float32),
        scratch_types=[
            pltpu.VMEM((CHUNK,), jnp.int32),
            pltpu.VMEM((CHUNK, D), jnp.float32),
            pltpu.VMEM((CHUNK, D), jnp.float32),
            pltpu.SemaphoreType.DMA,
        ],
    )
    out_rows = run(x_rows, idx, table)
    return out_rows.reshape(B, L, d_model)


# speedup vs baseline: 1.1759x; 1.1759x over previous
import jax
import jax.numpy as jnp
from jax import lax
from jax.experimental import pallas as pl
from jax.experimental.pallas import tpu as pltpu
from jax.experimental.pallas import tpu_sc as plsc

D = 512
LANES = 16
CHUNK = 32
NBUF = 3


def _pe_add_body(x_hbm, idx_hbm, table_hbm, out_hbm,
                 idx_v, xb0, xb1, xb2, pb0, pb1, pb2,
                 sx0, sx1, sx2, sg0, sg1, sg2, sw0, sw1, sw2):
    nc = 2
    wid = lax.axis_index("s") * nc + lax.axis_index("c")
    n_rows = x_hbm.shape[0]
    rows_per_worker = n_rows // 32
    n_chunks = rows_per_worker // CHUNK
    w0 = wid * rows_per_worker

    xbufs = [xb0, xb1, xb2]
    pbufs = [pb0, pb1, pb2]
    sx = [sx0, sx1, sx2]
    sg = [sg0, sg1, sg2]
    sw = [sw0, sw1, sw2]

    pltpu.sync_copy(idx_hbm.at[pl.ds(w0, rows_per_worker)], idx_v)

    xloads = [None] * n_chunks
    gathers = [None] * n_chunks
    wbs = [None] * n_chunks
    for t in range(n_chunks + 1):
        if t < n_chunks:
            b = t % NBUF
            if t >= NBUF:
                wbs[t - NBUF].wait()
            xloads[t] = pltpu.async_copy(
                x_hbm.at[pl.ds(w0 + t * CHUNK, CHUNK)], xbufs[b], sx[b])
            gathers[t] = pltpu.async_copy(
                table_hbm.at[idx_v.at[pl.ds(t * CHUNK, CHUNK)]],
                pbufs[b], sg[b])
        c = t - 1
        if c >= 0:
            b = c % NBUF
            xloads[c].wait()
            gathers[c].wait()
            xv, pv = xbufs[b], pbufs[b]

            def row_body(r, carry, xv=xv, pv=pv):
                for j in range(D // LANES):
                    sl = pl.ds(j * LANES, LANES)
                    xv[r, sl] = xv[r, sl] + pv[r, sl]
                return carry

            lax.fori_loop(0, CHUNK, row_body, 0)
            wbs[c] = pltpu.async_copy(
                xbufs[b], out_hbm.at[pl.ds(w0 + c * CHUNK, CHUNK)], sw[b])
    for c in range(n_chunks - NBUF, n_chunks):
        wbs[c].wait()


def kernel(x, pos, img_pe, seq_pe):
    B, L, d_model = x.shape
    n_rows = B * L * 2
    x_rows = x.reshape(n_rows, D)
    table = jnp.concatenate([img_pe, seq_pe], axis=0)
    idx = (pos.astype(jnp.int32) + jnp.array([0, img_pe.shape[0]], jnp.int32)
           ).reshape(n_rows)

    mesh = plsc.VectorSubcoreMesh(core_axis_name="c", subcore_axis_name="s")
    run = pl.kernel(
        _pe_add_body,
        mesh=mesh,
        out_type=jax.ShapeDtypeStruct((n_rows, D), jnp.float32),
        scratch_types=(
            [pltpu.VMEM((n_rows // 32,), jnp.int32)]
            + [pltpu.VMEM((CHUNK, D), jnp.float32) for _ in range(2 * NBUF)]
            + [pltpu.SemaphoreType.DMA for _ in range(3 * NBUF)]
        ),
    )
    out_rows = run(x_rows, idx, table)
    return out_rows.reshape(B, L, d_model)


# speedup vs baseline: 1.5651x; 1.3310x over previous
import jax
import jax.numpy as jnp
from jax import lax
from jax.experimental import pallas as pl
from jax.experimental.pallas import tpu as pltpu
from jax.experimental.pallas import tpu_sc as plsc

D = 512
LANES = 16
CR = 8
NBUF = 4
NW = 32


def _pe_add_body(x_hbm, idx_hbm, table_hbm, out_hbm,
                 idx_v, xb0, xb1, xb2, xb3, pb0, pb1, pb2, pb3,
                 sx0, sx1, sx2, sx3, sg0, sg1, sg2, sg3,
                 sw0, sw1, sw2, sw3):
    nc = 2
    wid = lax.axis_index("s") * nc + lax.axis_index("c")
    B, L, _ = x_hbm.shape
    rows_w = (B * L) // NW
    wpb = L // rows_w
    b_idx = wid // wpb
    l0 = (wid % wpb) * rows_w
    i0 = wid * 2 * rows_w
    n_chunks = rows_w // CR
    n_groups = n_chunks // NBUF

    xbufs = [xb0, xb1, xb2, xb3]
    pbufs = [pb0, pb1, pb2, pb3]
    sx = [sx0, sx1, sx2, sx3]
    sg = [sg0, sg1, sg2, sg3]
    sw = [sw0, sw1, sw2, sw3]

    pltpu.sync_copy(idx_hbm.at[pl.ds(i0, 2 * rows_w)], idx_v)

    def issue(c, b):
        pltpu.async_copy(
            x_hbm.at[b_idx, pl.ds(l0 + c * CR, CR)], xbufs[b], sx[b])
        pltpu.async_copy(
            table_hbm.at[idx_v.at[pl.ds(c * 2 * CR, 2 * CR)]],
            pbufs[b], sg[b])

    def process(c, b):
        pltpu.make_async_copy(
            x_hbm.at[b_idx, pl.ds(l0, CR)], xbufs[b], sx[b]).wait()
        pltpu.make_async_copy(
            table_hbm.at[pl.ds(0, 2 * CR)], pbufs[b], sg[b]).wait()
        xv, pv = xbufs[b], pbufs[b]

        def half_row(p, carry):
            r = p // 2
            col0 = (p % 2) * D
            for j in range(D // LANES):
                xs = pl.ds(col0 + j * LANES, LANES)
                xv[r, xs] = xv[r, xs] + pv[p, pl.ds(j * LANES, LANES)]
            return carry

        lax.fori_loop(0, 2 * CR, half_row, 0)
        pltpu.async_copy(
            xbufs[b], out_hbm.at[b_idx, pl.ds(l0 + c * CR, CR)], sw[b])

    def wait_wb(b):
        pltpu.make_async_copy(
            xbufs[b], out_hbm.at[b_idx, pl.ds(l0, CR)], sw[b]).wait()

    issue(0, 0)
    for k in range(1, NBUF):
        issue(k, k)
        process(k - 1, k - 1)

    def group(g, carry):
        for k in range(NBUF):
            t = g * NBUF + k
            wait_wb(k)
            issue(t, k)
            process(t - 1, (k + NBUF - 1) % NBUF)
        return carry

    lax.fori_loop(1, n_groups, group, 0)

    process(n_chunks - 1, (n_chunks - 1) % NBUF)
    for k in range(NBUF):
        wait_wb(k)


def kernel(x, pos, img_pe, seq_pe):
    B, L, d_model = x.shape
    table = jnp.concatenate([img_pe, seq_pe], axis=0)
    idx = (pos.astype(jnp.int32) + jnp.array([0, img_pe.shape[0]], jnp.int32)
           ).reshape(B * L * 2)

    mesh = plsc.VectorSubcoreMesh(core_axis_name="c", subcore_axis_name="s")
    run = pl.kernel(
        _pe_add_body,
        mesh=mesh,
        out_type=jax.ShapeDtypeStruct((B, L, d_model), jnp.float32),
        scratch_types=(
            [pltpu.VMEM((2 * B * L // NW,), jnp.int32)]
            + [pltpu.VMEM((CR, 2 * D), jnp.float32) for _ in range(NBUF)]
            + [pltpu.VMEM((2 * CR, D), jnp.float32) for _ in range(NBUF)]
            + [pltpu.SemaphoreType.DMA for _ in range(3 * NBUF)]
        ),
    )
    return run(x, idx, table)
